# baseline (speedup 1.0000x reference)
import functools

import jax
import jax.numpy as jnp
from jax import lax
from jax.experimental import pallas as pl
from jax.experimental.pallas import tpu as pltpu
from jax.experimental.pallas import tpu_sc as plsc


@functools.cache
def _build(B, D):
    info = plsc.get_sparse_core_info()
    NC, NS = info.num_cores, info.num_subcores
    NW = NC * NS
    assert B % (8 * NW) == 0
    b_per_w = B // NW
    mesh = plsc.VectorSubcoreMesh(core_axis_name="c", subcore_axis_name="s")

    row = jax.ShapeDtypeStruct((B, D), jnp.float32)

    @functools.partial(
        pl.kernel,
        mesh=mesh,
        out_type=(row, row, row),
        compiler_params=pltpu.CompilerParams(use_tc_tiling_on_sc=False),
        scratch_types=[
            pltpu.VMEM((b_per_w,), jnp.int32),
            pltpu.VMEM((b_per_w,), jnp.int32),
            pltpu.VMEM((b_per_w,), jnp.int32),
            pltpu.VMEM((b_per_w, D), jnp.float32),
            pltpu.VMEM((b_per_w, D), jnp.float32),
            pltpu.VMEM((b_per_w, D), jnp.float32),
            pltpu.SemaphoreType.DMA,
            pltpu.SemaphoreType.DMA,
            pltpu.SemaphoreType.DMA,
            pltpu.SemaphoreType.DMA,
        ],
    )
    def k(u_hbm, p_hbm, n_hbm, utab_hbm, itab_hbm, out_u, out_p, out_n,
          iu, ip, inn, ru, rp, rn, su, sp, sn, wsem):
        wid = lax.axis_index("s") * NC + lax.axis_index("c")
        base = wid * b_per_w
        sl = pl.ds(base, b_per_w)
        pltpu.sync_copy(u_hbm.at[sl], iu)
        pltpu.sync_copy(p_hbm.at[sl], ip)
        pltpu.sync_copy(n_hbm.at[sl], inn)
        cu = pltpu.async_copy(utab_hbm.at[iu], ru, su)
        cp = pltpu.async_copy(itab_hbm.at[ip], rp, sp)
        cn = pltpu.async_copy(itab_hbm.at[inn], rn, sn)
        cu.wait()
        wu = pltpu.async_copy(ru, out_u.at[sl], wsem)
        cp.wait()
        wp = pltpu.async_copy(rp, out_p.at[sl], wsem)
        cn.wait()
        wn = pltpu.async_copy(rn, out_n.at[sl], wsem)
        wu.wait()
        wp.wait()
        wn.wait()

    return k


def kernel(users, pos_items, neg_items, user_table, item_table):
    B = users.shape[0]
    D = user_table.shape[1]
    k = _build(B, D)
    return k(
        users.astype(jnp.int32),
        pos_items.astype(jnp.int32),
        neg_items.astype(jnp.int32),
        user_table,
        item_table,
    )
